# Initial kernel scaffold; baseline (speedup 1.0000x reference)
import jax
import jax.numpy as jnp
from jax.experimental import pallas as pl


def kernel(x, s_params, W1, b1, W2, b2, head_w, head_b, rows, cols, comp):
    raise NotImplementedError("write your pallas kernel here")


# speedup vs baseline: 91.3851x; 91.3851x over previous
import functools

import jax
import jax.numpy as jnp
from jax import lax
from jax.experimental import pallas as pl
from jax.experimental.pallas import tpu as pltpu
from jax.experimental.pallas import tpu_sc as plsc

N = 10000
T = 8
E_HALF = 160000
ES = 2 * E_HALF
NT = N * T

NC, NS = 2, 16
NW = NC * NS
E_TILE = ES // NW
KB = 125
N_IT = E_TILE // KB
N_PAD = 10240
ROWS_PER_TILE = N_PAD // NS

NBLK = 1000
RB = 2000


@functools.cache
def _make_sc_spmm(wc):
    mesh = plsc.VectorSubcoreMesh(core_axis_name="c", subcore_axis_name="s")

    @functools.partial(
        pl.kernel,
        out_type=jax.ShapeDtypeStruct((NC * N_PAD, wc), jnp.float32),
        mesh=mesh,
        scratch_types=[
            pltpu.VMEM((N_IT, KB), jnp.int32),
            pltpu.VMEM((N_IT, KB), jnp.int32),
            pltpu.VMEM((KB, wc), jnp.float32),
            pltpu.VMEM_SHARED((N_PAD, wc), jnp.float32),
            pltpu.SemaphoreType.DMA,
        ],
    )
    def spmm(m_hbm, src_hbm, dst_hbm, zero_hbm, out_hbm,
             src_v, dst_v, buf, acc, sem):
        cid = lax.axis_index("c")
        sid = lax.axis_index("s")
        wid = sid * NC + cid
        pltpu.sync_copy(src_hbm.at[wid], src_v)
        pltpu.sync_copy(dst_hbm.at[wid], dst_v)
        pltpu.sync_copy(zero_hbm, acc.at[pl.ds(sid * ROWS_PER_TILE,
                                               ROWS_PER_TILE)])
        plsc.subcore_barrier()

        def body(it, carry):
            pltpu.async_copy(m_hbm.at[dst_v.at[it]], buf, sem).wait()
            pltpu.sync_copy(buf, acc.at[src_v.at[it]], add=True)
            return carry

        lax.fori_loop(0, N_IT, body, 0)
        plsc.subcore_barrier()
        base = cid * N_PAD + sid * ROWS_PER_TILE
        pltpu.sync_copy(acc.at[pl.ds(sid * ROWS_PER_TILE, ROWS_PER_TILE)],
                        out_hbm.at[pl.ds(base, ROWS_PER_TILE)])

    return spmm


def _sc_spmm(m2, src_r, dst_r):
    w = m2.shape[1]
    fn = _make_sc_spmm(128)
    zero = jnp.zeros((ROWS_PER_TILE, 128), jnp.float32)
    if w <= 128:
        m2p = m2 if w == 128 else jnp.pad(m2, ((0, 0), (0, 128 - w)))
        part = fn(m2p, src_r, dst_r, zero)
        return part.reshape(NC, N_PAD, 128)[:, :N, :w].sum(axis=0)
    assert w % 128 == 0
    nch = w // 128
    mt = jnp.transpose(m2.reshape(N, nch, 128), (1, 0, 2))
    outs = []
    for c in range(nch):
        part = fn(mt[c], src_r, dst_r, zero)
        outs.append(part.reshape(NC, N_PAD, 128)[:, :N].sum(axis=0))
    sm = jnp.stack(outs, axis=0)
    return jnp.transpose(sm, (1, 0, 2)).reshape(N, w)


def _shifted(hs):
    z = jnp.zeros_like(hs[:, :, :1, :])
    return (jnp.concatenate([hs[:, :, 1:, :], z], axis=2)
            + jnp.concatenate([z, hs[:, :, :-1, :]], axis=2))


def _hop_pre(H, dis, s):
    n, b, t, f = H.shape

    def body(s_ref, h_ref, d_ref, m_ref):
        d4 = d_ref[...][:, None, :, None]
        hs = h_ref[...] * d4
        g = _shifted(hs)
        m_ref[...] = (s_ref[1] * hs + s_ref[3] * g).reshape(NBLK, b * t * f)

    return pl.pallas_call(
        body,
        grid=(n // NBLK,),
        in_specs=[
            pl.BlockSpec(memory_space=pltpu.SMEM),
            pl.BlockSpec((NBLK, b, t, f), lambda i: (i, 0, 0, 0)),
            pl.BlockSpec((NBLK, t), lambda i: (i, 0)),
        ],
        out_specs=pl.BlockSpec((NBLK, b * t * f), lambda i: (i, 0)),
        out_shape=jax.ShapeDtypeStruct((n, b * t * f), jnp.float32),
    )(s, H, dis)


def _hop_post(H, dis, sm, s):
    n, b, t, f = H.shape

    def body(s_ref, h_ref, d_ref, sm_ref, o_ref):
        d4 = d_ref[...][:, None, :, None]
        hs = h_ref[...] * d4
        g = _shifted(hs)
        u = s_ref[0] * hs + s_ref[2] * g
        o_ref[...] = d4 * (u + sm_ref[...].reshape(NBLK, b, t, f))

    return pl.pallas_call(
        body,
        grid=(n // NBLK,),
        in_specs=[
            pl.BlockSpec(memory_space=pltpu.SMEM),
            pl.BlockSpec((NBLK, b, t, f), lambda i: (i, 0, 0, 0)),
            pl.BlockSpec((NBLK, t), lambda i: (i, 0)),
            pl.BlockSpec((NBLK, b * t * f), lambda i: (i, 0)),
        ],
        out_specs=pl.BlockSpec((NBLK, b, t, f), lambda i: (i, 0, 0, 0)),
        out_shape=jax.ShapeDtypeStruct(H.shape, jnp.float32),
    )(s, H, dis, sm)


def _mma(x2, wk, acc, bias, do_relu):
    r, f = x2.shape
    fo = wk.shape[1]
    with_acc = acc is not None

    def body(x_ref, w_ref, a_ref, o_ref):
        xv = x_ref[...]
        if f == 1:
            prod = xv * w_ref[...]
        else:
            prod = jnp.dot(xv, w_ref[...], preferred_element_type=jnp.float32)
        out = prod + a_ref[...]
        if do_relu:
            out = jnp.maximum(out, 0.0)
        o_ref[...] = out

    if with_acc:
        a_arr = acc
        a_spec = pl.BlockSpec((RB, fo), lambda i: (i, 0))
    else:
        a_arr = bias.reshape(1, fo)
        a_spec = pl.BlockSpec((1, fo), lambda i: (0, 0))

    return pl.pallas_call(
        body,
        grid=(r // RB,),
        in_specs=[
            pl.BlockSpec((RB, f), lambda i: (i, 0)),
            pl.BlockSpec((f, fo), lambda i: (0, 0)),
            a_spec,
        ],
        out_specs=pl.BlockSpec((RB, fo), lambda i: (i, 0)),
        out_shape=jax.ShapeDtypeStruct((r, fo), jnp.float32),
    )(x2, wk, a_arr)


def _head(H, head_w, head_b):
    n, b, t, f = H.shape

    def body(h_ref, w_ref, hb_ref, o_ref):
        hm = jnp.mean(h_ref[...], axis=2)
        y = jnp.dot(hm.reshape(NBLK * b, f), w_ref[...],
                    preferred_element_type=jnp.float32)
        o_ref[...] = y.reshape(NBLK, b) + hb_ref[0]

    return pl.pallas_call(
        body,
        grid=(n // NBLK,),
        in_specs=[
            pl.BlockSpec((NBLK, b, t, f), lambda i: (i, 0, 0, 0)),
            pl.BlockSpec((f, 1), lambda i: (0, 0)),
            pl.BlockSpec(memory_space=pltpu.SMEM),
        ],
        out_specs=pl.BlockSpec((NBLK, b), lambda i: (i, 0)),
        out_shape=jax.ShapeDtypeStruct((n, b), jnp.float32),
    )(H, head_w, head_b)


def _gconv_layer(H, W, bvec, dis, s, src_r, dst_r):
    n, b, t, f = H.shape
    acc = _mma(H.reshape(n * b * t, f), W[0], None, bvec, False)
    Hk = H
    for k in range(1, W.shape[0]):
        m2 = _hop_pre(Hk, dis, s)
        sm = _sc_spmm(m2, src_r, dst_r)
        Hk = _hop_post(Hk, dis, sm, s)
        acc = _mma(Hk.reshape(n * b * t, f), W[k], acc, None,
                   k == W.shape[0] - 1)
    return acc.reshape(n, b, t, W.shape[2])


def kernel(x, s_params, W1, b1, W2, b2, head_w, head_b, rows, cols, comp):
    del comp
    bsz = x.shape[0]
    s_src = lax.slice(rows, (NT,), (NT + ES,))
    s_dst = lax.slice(cols, (NT,), (NT + ES,))
    src_r = s_src.reshape(NW, N_IT, KB)
    dst_r = s_dst.reshape(NW, N_IT, KB)
    s = jax.nn.relu(s_params)

    d_out = _sc_spmm(jnp.ones((N, 16), jnp.float32), src_r, dst_r)
    dS = d_out[:, 0]
    tt = jnp.arange(T)
    dT = jnp.where((tt == 0) | (tt == T - 1), 1.0, 2.0)
    deg = (s[0] + s[2] * dT)[None, :] + dS[:, None] * (s[1] + s[3] * dT)[None, :]
    dis = jnp.where(deg > 0, lax.rsqrt(jnp.where(deg > 0, deg, 1.0)), 0.0)

    H0 = jnp.transpose(x, (2, 0, 3, 1))
    H1 = _gconv_layer(H0, W1, b1, dis, s, src_r, dst_r)
    H2 = _gconv_layer(H1, W2, b2, dis, s, src_r, dst_r)
    y = _head(H2, head_w, head_b)
    return y.T.reshape(bsz, N)


# speedup vs baseline: 120.0415x; 1.3136x over previous
import functools

import jax
import jax.numpy as jnp
from jax import lax
from jax.experimental import pallas as pl
from jax.experimental.pallas import tpu as pltpu
from jax.experimental.pallas import tpu_sc as plsc

N = 10000
T = 8
E_HALF = 160000
ES = 2 * E_HALF
NT = N * T

NC, NS = 2, 16
NW = NC * NS
E_TILE = ES // NW
KB = 125
N_IT = E_TILE // KB
N_HALVES = 2
HN = N_IT // N_HALVES
N_PAD = 10240
ROWS_PER_TILE = N_PAD // NS

NBLK = 1000
RB = 2000


@functools.cache
def _make_sc_spmm(wc):
    mesh = plsc.VectorSubcoreMesh(core_axis_name="c", subcore_axis_name="s")

    @functools.partial(
        pl.kernel,
        out_type=jax.ShapeDtypeStruct((NC * N_PAD, wc), jnp.float32),
        mesh=mesh,
        scratch_types=[
            pltpu.VMEM((HN, KB), jnp.int32),
            pltpu.VMEM((HN, KB), jnp.int32),
            pltpu.VMEM((2, KB, wc), jnp.float32),
            pltpu.VMEM_SHARED((N_PAD, wc), jnp.float32),
            pltpu.SemaphoreType.DMA,
            pltpu.SemaphoreType.DMA,
        ],
    )
    def spmm(m_hbm, src_hbm, dst_hbm, zero_hbm, out_hbm,
             src_v, dst_v, buf2, acc, sem0, sem1):
        cid = lax.axis_index("c")
        sid = lax.axis_index("s")
        wid = sid * NC + cid
        pltpu.sync_copy(zero_hbm, acc.at[pl.ds(sid * ROWS_PER_TILE,
                                               ROWS_PER_TILE)])
        plsc.subcore_barrier()

        for h in range(N_HALVES):
            pltpu.sync_copy(src_hbm.at[wid].at[h], src_v)
            pltpu.sync_copy(dst_hbm.at[wid].at[h], dst_v)
            pltpu.async_copy(m_hbm.at[dst_v.at[0]], buf2.at[0], sem0)
            pltpu.async_copy(m_hbm.at[dst_v.at[1]], buf2.at[1], sem1)

            def wait(sem):
                pltpu.make_async_copy(m_hbm.at[dst_v.at[0]], buf2.at[0],
                                      sem).wait()

            def body(i, carry):
                it = 2 * i
                wait(sem0)
                pltpu.sync_copy(buf2.at[0], acc.at[src_v.at[it]], add=True)
                pltpu.async_copy(m_hbm.at[dst_v.at[it + 2]], buf2.at[0], sem0)
                wait(sem1)
                pltpu.sync_copy(buf2.at[1], acc.at[src_v.at[it + 1]],
                                add=True)
                pltpu.async_copy(m_hbm.at[dst_v.at[it + 3]], buf2.at[1], sem1)
                return carry

            lax.fori_loop(0, HN // 2 - 1, body, 0)
            wait(sem0)
            pltpu.sync_copy(buf2.at[0], acc.at[src_v.at[HN - 2]], add=True)
            wait(sem1)
            pltpu.sync_copy(buf2.at[1], acc.at[src_v.at[HN - 1]], add=True)
        plsc.subcore_barrier()
        base = cid * N_PAD + sid * ROWS_PER_TILE
        pltpu.sync_copy(acc.at[pl.ds(sid * ROWS_PER_TILE, ROWS_PER_TILE)],
                        out_hbm.at[pl.ds(base, ROWS_PER_TILE)])

    return spmm


def _sc_spmm(m2, src_r, dst_r):
    w = m2.shape[1]
    fn = _make_sc_spmm(128)
    zero = jnp.zeros((ROWS_PER_TILE, 128), jnp.float32)
    if w <= 128:
        m2p = m2 if w == 128 else jnp.pad(m2, ((0, 0), (0, 128 - w)))
        part = fn(m2p, src_r, dst_r, zero)
        return part.reshape(NC, N_PAD, 128)[:, :N, :w].sum(axis=0)
    assert w % 128 == 0
    nch = w // 128
    mt = jnp.transpose(m2.reshape(N, nch, 128), (1, 0, 2))
    outs = []
    for c in range(nch):
        part = fn(mt[c], src_r, dst_r, zero)
        outs.append(part.reshape(NC, N_PAD, 128)[:, :N].sum(axis=0))
    sm = jnp.stack(outs, axis=0)
    return jnp.transpose(sm, (1, 0, 2)).reshape(N, w)


def _shifted(hs):
    z = jnp.zeros_like(hs[:, :, :1, :])
    return (jnp.concatenate([hs[:, :, 1:, :], z], axis=2)
            + jnp.concatenate([z, hs[:, :, :-1, :]], axis=2))


def _hop_pre(H, dis, s):
    n, b, t, f = H.shape

    def body(s_ref, h_ref, d_ref, m_ref):
        d4 = d_ref[...][:, None, :, None]
        hs = h_ref[...] * d4
        g = _shifted(hs)
        m_ref[...] = (s_ref[1] * hs + s_ref[3] * g).reshape(NBLK, b * t * f)

    return pl.pallas_call(
        body,
        grid=(n // NBLK,),
        in_specs=[
            pl.BlockSpec(memory_space=pltpu.SMEM),
            pl.BlockSpec((NBLK, b, t, f), lambda i: (i, 0, 0, 0)),
            pl.BlockSpec((NBLK, t), lambda i: (i, 0)),
        ],
        out_specs=pl.BlockSpec((NBLK, b * t * f), lambda i: (i, 0)),
        out_shape=jax.ShapeDtypeStruct((n, b * t * f), jnp.float32),
    )(s, H, dis)


def _hop_post(H, dis, sm, s):
    n, b, t, f = H.shape

    def body(s_ref, h_ref, d_ref, sm_ref, o_ref):
        d4 = d_ref[...][:, None, :, None]
        hs = h_ref[...] * d4
        g = _shifted(hs)
        u = s_ref[0] * hs + s_ref[2] * g
        o_ref[...] = d4 * (u + sm_ref[...].reshape(NBLK, b, t, f))

    return pl.pallas_call(
        body,
        grid=(n // NBLK,),
        in_specs=[
            pl.BlockSpec(memory_space=pltpu.SMEM),
            pl.BlockSpec((NBLK, b, t, f), lambda i: (i, 0, 0, 0)),
            pl.BlockSpec((NBLK, t), lambda i: (i, 0)),
            pl.BlockSpec((NBLK, b * t * f), lambda i: (i, 0)),
        ],
        out_specs=pl.BlockSpec((NBLK, b, t, f), lambda i: (i, 0, 0, 0)),
        out_shape=jax.ShapeDtypeStruct(H.shape, jnp.float32),
    )(s, H, dis, sm)


def _mma(x2, wk, acc, bias, do_relu):
    r, f = x2.shape
    fo = wk.shape[1]
    with_acc = acc is not None

    def body(x_ref, w_ref, a_ref, o_ref):
        xv = x_ref[...]
        if f == 1:
            prod = xv * w_ref[...]
        else:
            prod = jnp.dot(xv, w_ref[...], preferred_element_type=jnp.float32)
        out = prod + a_ref[...]
        if do_relu:
            out = jnp.maximum(out, 0.0)
        o_ref[...] = out

    if with_acc:
        a_arr = acc
        a_spec = pl.BlockSpec((RB, fo), lambda i: (i, 0))
    else:
        a_arr = bias.reshape(1, fo)
        a_spec = pl.BlockSpec((1, fo), lambda i: (0, 0))

    return pl.pallas_call(
        body,
        grid=(r // RB,),
        in_specs=[
            pl.BlockSpec((RB, f), lambda i: (i, 0)),
            pl.BlockSpec((f, fo), lambda i: (0, 0)),
            a_spec,
        ],
        out_specs=pl.BlockSpec((RB, fo), lambda i: (i, 0)),
        out_shape=jax.ShapeDtypeStruct((r, fo), jnp.float32),
    )(x2, wk, a_arr)


def _head(H, head_w, head_b):
    n, b, t, f = H.shape

    def body(h_ref, w_ref, hb_ref, o_ref):
        hm = jnp.mean(h_ref[...], axis=2)
        y = jnp.dot(hm.reshape(NBLK * b, f), w_ref[...],
                    preferred_element_type=jnp.float32)
        o_ref[...] = y.reshape(NBLK, b) + hb_ref[0]

    return pl.pallas_call(
        body,
        grid=(n // NBLK,),
        in_specs=[
            pl.BlockSpec((NBLK, b, t, f), lambda i: (i, 0, 0, 0)),
            pl.BlockSpec((f, 1), lambda i: (0, 0)),
            pl.BlockSpec(memory_space=pltpu.SMEM),
        ],
        out_specs=pl.BlockSpec((NBLK, b), lambda i: (i, 0)),
        out_shape=jax.ShapeDtypeStruct((n, b), jnp.float32),
    )(H, head_w, head_b)


def _gconv_layer(H, W, bvec, dis, s, src_r, dst_r):
    n, b, t, f = H.shape
    acc = _mma(H.reshape(n * b * t, f), W[0], None, bvec, False)
    Hk = H
    for k in range(1, W.shape[0]):
        m2 = _hop_pre(Hk, dis, s)
        sm = _sc_spmm(m2, src_r, dst_r)
        Hk = _hop_post(Hk, dis, sm, s)
        acc = _mma(Hk.reshape(n * b * t, f), W[k], acc, None,
                   k == W.shape[0] - 1)
    return acc.reshape(n, b, t, W.shape[2])


def kernel(x, s_params, W1, b1, W2, b2, head_w, head_b, rows, cols, comp):
    del comp
    bsz = x.shape[0]
    s_src = lax.slice(rows, (NT,), (NT + ES,))
    s_dst = lax.slice(cols, (NT,), (NT + ES,))
    src_r = s_src.reshape(NW, N_HALVES, HN, KB)
    dst_r = s_dst.reshape(NW, N_HALVES, HN, KB)
    s = jax.nn.relu(s_params)

    d_out = _sc_spmm(jnp.ones((N, 16), jnp.float32), src_r, dst_r)
    dS = d_out[:, 0]
    tt = jnp.arange(T)
    dT = jnp.where((tt == 0) | (tt == T - 1), 1.0, 2.0)
    deg = (s[0] + s[2] * dT)[None, :] + dS[:, None] * (s[1] + s[3] * dT)[None, :]
    dis = jnp.where(deg > 0, lax.rsqrt(jnp.where(deg > 0, deg, 1.0)), 0.0)

    H0 = jnp.transpose(x, (2, 0, 3, 1))
    H1 = _gconv_layer(H0, W1, b1, dis, s, src_r, dst_r)
    H2 = _gconv_layer(H1, W2, b2, dis, s, src_r, dst_r)
    y = _head(H2, head_w, head_b)
    return y.T.reshape(bsz, N)


# speedup vs baseline: 134.4201x; 1.1198x over previous
import functools

import jax
import jax.numpy as jnp
from jax import lax
from jax.experimental import pallas as pl
from jax.experimental.pallas import tpu as pltpu
from jax.experimental.pallas import tpu_sc as plsc

N = 10000
T = 8
E_HALF = 160000
ES = 2 * E_HALF
NT = N * T

NC, NS = 2, 16
NW = NC * NS
E_TILE = ES // NW
KB = 125
N_IT = E_TILE // KB
N_HALVES = 2
HN = N_IT // N_HALVES
N_PAD = 10240
ROWS_PER_TILE = N_PAD // NS

NBLK = 1000
RB = 2000


@functools.cache
def _make_sc_spmm(nch):
    mesh = plsc.VectorSubcoreMesh(core_axis_name="c", subcore_axis_name="s")

    @functools.partial(
        pl.kernel,
        out_type=jax.ShapeDtypeStruct((NC, nch, N_PAD, 128), jnp.float32),
        mesh=mesh,
        scratch_types=[
            pltpu.VMEM((HN, KB), jnp.int32),
            pltpu.VMEM((HN, KB), jnp.int32),
            pltpu.VMEM((2, KB, 128), jnp.float32),
            pltpu.VMEM_SHARED((N_PAD, 128), jnp.float32),
            pltpu.SemaphoreType.DMA,
            pltpu.SemaphoreType.DMA,
        ],
    )
    def spmm(m_hbm, src_hbm, dst_hbm, zero_hbm, out_hbm,
             src_v, dst_v, buf2, acc, sem0, sem1):
        cid = lax.axis_index("c")
        sid = lax.axis_index("s")
        wid = sid * NC + cid
        my_rows = pl.ds(sid * ROWS_PER_TILE, ROWS_PER_TILE)

        def chunk_body(c, carry):
            pltpu.sync_copy(zero_hbm, acc.at[my_rows])
            plsc.subcore_barrier()
            mc = m_hbm.at[c]

            for h in range(N_HALVES):
                pltpu.sync_copy(src_hbm.at[wid].at[h], src_v)
                pltpu.sync_copy(dst_hbm.at[wid].at[h], dst_v)
                pltpu.async_copy(mc.at[dst_v.at[0]], buf2.at[0], sem0)
                pltpu.async_copy(mc.at[dst_v.at[1]], buf2.at[1], sem1)

                def wait(sem):
                    pltpu.make_async_copy(mc.at[dst_v.at[0]], buf2.at[0],
                                          sem).wait()

                def body(i, carry2):
                    it = 2 * i
                    wait(sem0)
                    pltpu.sync_copy(buf2.at[0], acc.at[src_v.at[it]],
                                    add=True)
                    pltpu.async_copy(mc.at[dst_v.at[it + 2]], buf2.at[0],
                                     sem0)
                    wait(sem1)
                    pltpu.sync_copy(buf2.at[1], acc.at[src_v.at[it + 1]],
                                    add=True)
                    pltpu.async_copy(mc.at[dst_v.at[it + 3]], buf2.at[1],
                                     sem1)
                    return carry2

                lax.fori_loop(0, HN // 2 - 1, body, 0)
                wait(sem0)
                pltpu.sync_copy(buf2.at[0], acc.at[src_v.at[HN - 2]],
                                add=True)
                wait(sem1)
                pltpu.sync_copy(buf2.at[1], acc.at[src_v.at[HN - 1]],
                                add=True)
            plsc.subcore_barrier()
            pltpu.sync_copy(acc.at[my_rows],
                            out_hbm.at[cid].at[c].at[my_rows])
            plsc.subcore_barrier()
            return carry

        lax.fori_loop(0, nch, chunk_body, 0)

    return spmm


def _sc_spmm(m2, src_r, dst_r):
    w = m2.shape[1]
    zero = jnp.zeros((ROWS_PER_TILE, 128), jnp.float32)
    if w <= 128:
        m2p = m2 if w == 128 else jnp.pad(m2, ((0, 0), (0, 128 - w)))
        part = _make_sc_spmm(1)(m2p[None], src_r, dst_r, zero)
        return part[:, 0, :N, :w].sum(axis=0)
    assert w % 128 == 0
    nch = w // 128
    mt = jnp.transpose(m2.reshape(N, nch, 128), (1, 0, 2))
    part = _make_sc_spmm(nch)(mt, src_r, dst_r, zero)
    sm = part[:, :, :N, :].sum(axis=0)
    return jnp.transpose(sm, (1, 0, 2)).reshape(N, w)


def _shifted(hs):
    z = jnp.zeros_like(hs[:, :, :1, :])
    return (jnp.concatenate([hs[:, :, 1:, :], z], axis=2)
            + jnp.concatenate([z, hs[:, :, :-1, :]], axis=2))


def _hop_pre(H, dis, s):
    n, b, t, f = H.shape

    def body(s_ref, h_ref, d_ref, m_ref):
        d4 = d_ref[...][:, None, :, None]
        hs = h_ref[...] * d4
        g = _shifted(hs)
        m_ref[...] = (s_ref[1] * hs + s_ref[3] * g).reshape(NBLK, b * t * f)

    return pl.pallas_call(
        body,
        grid=(n // NBLK,),
        in_specs=[
            pl.BlockSpec(memory_space=pltpu.SMEM),
            pl.BlockSpec((NBLK, b, t, f), lambda i: (i, 0, 0, 0)),
            pl.BlockSpec((NBLK, t), lambda i: (i, 0)),
        ],
        out_specs=pl.BlockSpec((NBLK, b * t * f), lambda i: (i, 0)),
        out_shape=jax.ShapeDtypeStruct((n, b * t * f), jnp.float32),
    )(s, H, dis)


def _hop_post(H, dis, sm, s):
    n, b, t, f = H.shape

    def body(s_ref, h_ref, d_ref, sm_ref, o_ref):
        d4 = d_ref[...][:, None, :, None]
        hs = h_ref[...] * d4
        g = _shifted(hs)
        u = s_ref[0] * hs + s_ref[2] * g
        o_ref[...] = d4 * (u + sm_ref[...].reshape(NBLK, b, t, f))

    return pl.pallas_call(
        body,
        grid=(n // NBLK,),
        in_specs=[
            pl.BlockSpec(memory_space=pltpu.SMEM),
            pl.BlockSpec((NBLK, b, t, f), lambda i: (i, 0, 0, 0)),
            pl.BlockSpec((NBLK, t), lambda i: (i, 0)),
            pl.BlockSpec((NBLK, b * t * f), lambda i: (i, 0)),
        ],
        out_specs=pl.BlockSpec((NBLK, b, t, f), lambda i: (i, 0, 0, 0)),
        out_shape=jax.ShapeDtypeStruct(H.shape, jnp.float32),
    )(s, H, dis, sm)


def _mma(x2, wk, acc, bias, do_relu):
    r, f = x2.shape
    fo = wk.shape[1]
    with_acc = acc is not None

    def body(x_ref, w_ref, a_ref, o_ref):
        xv = x_ref[...]
        if f == 1:
            prod = xv * w_ref[...]
        else:
            prod = jnp.dot(xv, w_ref[...], preferred_element_type=jnp.float32)
        out = prod + a_ref[...]
        if do_relu:
            out = jnp.maximum(out, 0.0)
        o_ref[...] = out

    if with_acc:
        a_arr = acc
        a_spec = pl.BlockSpec((RB, fo), lambda i: (i, 0))
    else:
        a_arr = bias.reshape(1, fo)
        a_spec = pl.BlockSpec((1, fo), lambda i: (0, 0))

    return pl.pallas_call(
        body,
        grid=(r // RB,),
        in_specs=[
            pl.BlockSpec((RB, f), lambda i: (i, 0)),
            pl.BlockSpec((f, fo), lambda i: (0, 0)),
            a_spec,
        ],
        out_specs=pl.BlockSpec((RB, fo), lambda i: (i, 0)),
        out_shape=jax.ShapeDtypeStruct((r, fo), jnp.float32),
    )(x2, wk, a_arr)


def _head(H, head_w, head_b):
    n, b, t, f = H.shape

    def body(h_ref, w_ref, hb_ref, o_ref):
        hm = jnp.mean(h_ref[...], axis=2)
        y = jnp.dot(hm.reshape(NBLK * b, f), w_ref[...],
                    preferred_element_type=jnp.float32)
        o_ref[...] = y.reshape(NBLK, b) + hb_ref[0]

    return pl.pallas_call(
        body,
        grid=(n // NBLK,),
        in_specs=[
            pl.BlockSpec((NBLK, b, t, f), lambda i: (i, 0, 0, 0)),
            pl.BlockSpec((f, 1), lambda i: (0, 0)),
            pl.BlockSpec(memory_space=pltpu.SMEM),
        ],
        out_specs=pl.BlockSpec((NBLK, b), lambda i: (i, 0)),
        out_shape=jax.ShapeDtypeStruct((n, b), jnp.float32),
    )(H, head_w, head_b)


def _gconv_layer(H, W, bvec, dis, s, src_r, dst_r):
    n, b, t, f = H.shape
    acc = _mma(H.reshape(n * b * t, f), W[0], None, bvec, False)
    Hk = H
    for k in range(1, W.shape[0]):
        m2 = _hop_pre(Hk, dis, s)
        sm = _sc_spmm(m2, src_r, dst_r)
        Hk = _hop_post(Hk, dis, sm, s)
        acc = _mma(Hk.reshape(n * b * t, f), W[k], acc, None,
                   k == W.shape[0] - 1)
    return acc.reshape(n, b, t, W.shape[2])


def kernel(x, s_params, W1, b1, W2, b2, head_w, head_b, rows, cols, comp):
    del comp
    bsz = x.shape[0]
    s_src = lax.slice(rows, (NT,), (NT + ES,))
    s_dst = lax.slice(cols, (NT,), (NT + ES,))
    src_r = s_src.reshape(NW, N_HALVES, HN, KB)
    dst_r = s_dst.reshape(NW, N_HALVES, HN, KB)
    s = jax.nn.relu(s_params)

    d_out = _sc_spmm(jnp.ones((N, 16), jnp.float32), src_r, dst_r)
    dS = d_out[:, 0]
    tt = jnp.arange(T)
    dT = jnp.where((tt == 0) | (tt == T - 1), 1.0, 2.0)
    deg = (s[0] + s[2] * dT)[None, :] + dS[:, None] * (s[1] + s[3] * dT)[None, :]
    dis = jnp.where(deg > 0, lax.rsqrt(jnp.where(deg > 0, deg, 1.0)), 0.0)

    H0 = jnp.transpose(x, (2, 0, 3, 1))
    H1 = _gconv_layer(H0, W1, b1, dis, s, src_r, dst_r)
    H2 = _gconv_layer(H1, W2, b2, dis, s, src_r, dst_r)
    y = _head(H2, head_w, head_b)
    return y.T.reshape(bsz, N)


# speedup vs baseline: 145.8788x; 1.0852x over previous
import functools

import jax
import jax.numpy as jnp
from jax import lax
from jax.experimental import pallas as pl
from jax.experimental.pallas import tpu as pltpu
from jax.experimental.pallas import tpu_sc as plsc

N = 10000
T = 8
E_HALF = 160000
ES = 2 * E_HALF
NT = N * T

NC, NS = 2, 16
NW = NC * NS
E_TILE = ES // NW
KB = 125
N_IT = E_TILE // KB
N_HALVES = 2
HN = N_IT // N_HALVES
N_PAD = 10240
ROWS_PER_TILE = N_PAD // NS

NBLK = 1000
HBLK = 400
RB = 2000


@functools.cache
def _make_sc_spmm(nch):
    mesh = plsc.VectorSubcoreMesh(core_axis_name="c", subcore_axis_name="s")

    @functools.partial(
        pl.kernel,
        out_type=jax.ShapeDtypeStruct((NC, nch, N_PAD, 128), jnp.float32),
        mesh=mesh,
        scratch_types=[
            pltpu.VMEM((HN, KB), jnp.int32),
            pltpu.VMEM((HN, KB), jnp.int32),
            pltpu.VMEM((2, KB, 128), jnp.float32),
            pltpu.VMEM_SHARED((N_PAD, 128), jnp.float32),
            pltpu.SemaphoreType.DMA,
            pltpu.SemaphoreType.DMA,
        ],
    )
    def spmm(m_hbm, src_hbm, dst_hbm, zero_hbm, out_hbm,
             src_v, dst_v, buf2, acc, sem0, sem1):
        cid = lax.axis_index("c")
        sid = lax.axis_index("s")
        wid = sid * NC + cid
        my_rows = pl.ds(sid * ROWS_PER_TILE, ROWS_PER_TILE)

        def chunk_body(c, carry):
            pltpu.sync_copy(zero_hbm, acc.at[my_rows])
            plsc.subcore_barrier()
            mc = m_hbm.at[c]

            for h in range(N_HALVES):
                pltpu.sync_copy(src_hbm.at[wid].at[h], src_v)
                pltpu.sync_copy(dst_hbm.at[wid].at[h], dst_v)
                pltpu.async_copy(mc.at[dst_v.at[0]], buf2.at[0], sem0)
                pltpu.async_copy(mc.at[dst_v.at[1]], buf2.at[1], sem1)

                def wait(sem):
                    pltpu.make_async_copy(mc.at[dst_v.at[0]], buf2.at[0],
                                          sem).wait()

                def body(i, carry2):
                    it = 2 * i
                    wait(sem0)
                    pltpu.sync_copy(buf2.at[0], acc.at[src_v.at[it]],
                                    add=True)
                    pltpu.async_copy(mc.at[dst_v.at[it + 2]], buf2.at[0],
                                     sem0)
                    wait(sem1)
                    pltpu.sync_copy(buf2.at[1], acc.at[src_v.at[it + 1]],
                                    add=True)
                    pltpu.async_copy(mc.at[dst_v.at[it + 3]], buf2.at[1],
                                     sem1)
                    return carry2

                lax.fori_loop(0, HN // 2 - 1, body, 0)
                wait(sem0)
                pltpu.sync_copy(buf2.at[0], acc.at[src_v.at[HN - 2]],
                                add=True)
                wait(sem1)
                pltpu.sync_copy(buf2.at[1], acc.at[src_v.at[HN - 1]],
                                add=True)
            plsc.subcore_barrier()
            pltpu.sync_copy(acc.at[my_rows],
                            out_hbm.at[cid].at[c].at[my_rows])
            plsc.subcore_barrier()
            return carry

        lax.fori_loop(0, nch, chunk_body, 0)

    return spmm


def _sc_spmm(mt, src_r, dst_r):
    zero = jnp.zeros((ROWS_PER_TILE, 128), jnp.float32)
    return _make_sc_spmm(mt.shape[0])(mt, src_r, dst_r, zero)


def _shifted(hs):
    z = jnp.zeros_like(hs[:, :, :1, :])
    return (jnp.concatenate([hs[:, :, 1:, :], z], axis=2)
            + jnp.concatenate([z, hs[:, :, :-1, :]], axis=2))


def _hop_pre(H, dis, s):
    n, b, t, f = H.shape
    w = b * t * f
    nch = max(1, w // 128)
    blk = HBLK if w >= 128 else NBLK

    def body(s_ref, h_ref, d_ref, m_ref):
        d4 = d_ref[...][:, None, :, None]
        hs = h_ref[...] * d4
        g = _shifted(hs)
        m = (s_ref[1] * hs + s_ref[3] * g).reshape(blk, w)
        if w < 128:
            m_ref[0] = jnp.pad(m, ((0, 0), (0, 128 - w)))
        else:
            for c in range(nch):
                m_ref[c] = m[:, c * 128:(c + 1) * 128]

    return pl.pallas_call(
        body,
        grid=(n // blk,),
        in_specs=[
            pl.BlockSpec(memory_space=pltpu.SMEM),
            pl.BlockSpec((blk, b, t, f), lambda i: (i, 0, 0, 0)),
            pl.BlockSpec((blk, t), lambda i: (i, 0)),
        ],
        out_specs=pl.BlockSpec((nch, blk, 128), lambda i: (0, i, 0)),
        out_shape=jax.ShapeDtypeStruct((nch, n, 128), jnp.float32),
    )(s, H, dis)


def _hop_post(H, dis, part, s):
    n, b, t, f = H.shape
    w = b * t * f
    wide = w >= 128
    nch = max(1, w // 128)
    blk = HBLK if wide else NBLK

    def body(s_ref, h_ref, d_ref, sm_ref, o_ref):
        d4 = d_ref[...][:, None, :, None]
        hs = h_ref[...] * d4
        g = _shifted(hs)
        u = s_ref[0] * hs + s_ref[2] * g
        if wide:
            smv = sm_ref[...]
            smc = smv[0] + smv[1]
            sm = jnp.concatenate([smc[c] for c in range(nch)], axis=-1)
            sm = sm.reshape(blk, b, t, f)
        else:
            sm = sm_ref[...].reshape(blk, b, t, f)
        o_ref[...] = d4 * (u + sm)

    if wide:
        sm_spec = pl.BlockSpec((NC, nch, blk, 128), lambda i: (0, 0, i, 0))
    else:
        sm_spec = pl.BlockSpec((blk, w), lambda i: (i, 0))
        part = part[:, 0, :N, :w].sum(axis=0)

    return pl.pallas_call(
        body,
        grid=(n // blk,),
        in_specs=[
            pl.BlockSpec(memory_space=pltpu.SMEM),
            pl.BlockSpec((blk, b, t, f), lambda i: (i, 0, 0, 0)),
            pl.BlockSpec((blk, t), lambda i: (i, 0)),
            sm_spec,
        ],
        out_specs=pl.BlockSpec((blk, b, t, f), lambda i: (i, 0, 0, 0)),
        out_shape=jax.ShapeDtypeStruct(H.shape, jnp.float32),
    )(s, H, dis, part)


def _mma(x2, wk, acc, bias, do_relu):
    r, f = x2.shape
    fo = wk.shape[1]
    with_acc = acc is not None

    def body(x_ref, w_ref, a_ref, o_ref):
        xv = x_ref[...]
        if f == 1:
            prod = xv * w_ref[...]
        else:
            prod = jnp.dot(xv, w_ref[...], preferred_element_type=jnp.float32)
        out = prod + a_ref[...]
        if do_relu:
            out = jnp.maximum(out, 0.0)
        o_ref[...] = out

    if with_acc:
        a_arr = acc
        a_spec = pl.BlockSpec((RB, fo), lambda i: (i, 0))
    else:
        a_arr = bias.reshape(1, fo)
        a_spec = pl.BlockSpec((1, fo), lambda i: (0, 0))

    return pl.pallas_call(
        body,
        grid=(r // RB,),
        in_specs=[
            pl.BlockSpec((RB, f), lambda i: (i, 0)),
            pl.BlockSpec((f, fo), lambda i: (0, 0)),
            a_spec,
        ],
        out_specs=pl.BlockSpec((RB, fo), lambda i: (i, 0)),
        out_shape=jax.ShapeDtypeStruct((r, fo), jnp.float32),
    )(x2, wk, a_arr)


def _head(H, head_w, head_b):
    n, b, t, f = H.shape

    def body(h_ref, w_ref, hb_ref, o_ref):
        hm = jnp.mean(h_ref[...], axis=2)
        y = jnp.dot(hm.reshape(NBLK * b, f), w_ref[...],
                    preferred_element_type=jnp.float32)
        o_ref[...] = y.reshape(NBLK, b) + hb_ref[0]

    return pl.pallas_call(
        body,
        grid=(n // NBLK,),
        in_specs=[
            pl.BlockSpec((NBLK, b, t, f), lambda i: (i, 0, 0, 0)),
            pl.BlockSpec((f, 1), lambda i: (0, 0)),
            pl.BlockSpec(memory_space=pltpu.SMEM),
        ],
        out_specs=pl.BlockSpec((NBLK, b), lambda i: (i, 0)),
        out_shape=jax.ShapeDtypeStruct((n, b), jnp.float32),
    )(H, head_w, head_b)


def _gconv_layer(H, W, bvec, dis, s, src_r, dst_r):
    n, b, t, f = H.shape
    acc = _mma(H.reshape(n * b * t, f), W[0], None, bvec, False)
    Hk = H
    for k in range(1, W.shape[0]):
        mt = _hop_pre(Hk, dis, s)
        part = _sc_spmm(mt, src_r, dst_r)
        Hk = _hop_post(Hk, dis, part, s)
        acc = _mma(Hk.reshape(n * b * t, f), W[k], acc, None,
                   k == W.shape[0] - 1)
    return acc.reshape(n, b, t, W.shape[2])


def kernel(x, s_params, W1, b1, W2, b2, head_w, head_b, rows, cols, comp):
    del comp
    bsz = x.shape[0]
    s_src = lax.slice(rows, (NT,), (NT + ES,))
    s_dst = lax.slice(cols, (NT,), (NT + ES,))
    src_r = s_src.reshape(NW, N_HALVES, HN, KB)
    dst_r = s_dst.reshape(NW, N_HALVES, HN, KB)
    s = jax.nn.relu(s_params)

    d_part = _sc_spmm(jnp.ones((1, N, 128), jnp.float32), src_r, dst_r)
    dS = d_part[:, 0, :N, 0].sum(axis=0)
    tt = jnp.arange(T)
    dT = jnp.where((tt == 0) | (tt == T - 1), 1.0, 2.0)
    deg = (s[0] + s[2] * dT)[None, :] + dS[:, None] * (s[1] + s[3] * dT)[None, :]
    dis = jnp.where(deg > 0, lax.rsqrt(jnp.where(deg > 0, deg, 1.0)), 0.0)

    H0 = jnp.transpose(x, (2, 0, 3, 1))
    H1 = _gconv_layer(H0, W1, b1, dis, s, src_r, dst_r)
    H2 = _gconv_layer(H1, W2, b2, dis, s, src_r, dst_r)
    y = _head(H2, head_w, head_b)
    return y.T.reshape(bsz, N)
